# Initial kernel scaffold; baseline (speedup 1.0000x reference)
import jax
import jax.numpy as jnp
from jax.experimental import pallas as pl


def kernel(x):
    raise NotImplementedError("write your pallas kernel here")


# speedup vs baseline: 5.3115x; 5.3115x over previous
import functools

import jax
import jax.numpy as jnp
from jax.experimental import pallas as pl

K = 12
LOG2 = 0.6931471805599453


def _span_body(x_ref, out_ref, *, bl: int):
    i = pl.program_id(0)
    base = i * bl
    w = x_ref[pl.ds(base, bl + 16), :]
    a = w[:bl]
    out_ref[:, 0, :] = a + LOG2
    for k in range(1, K):
        b = jax.lax.slice_in_dim(w, k, k + bl, axis=0)
        out_ref[:, k, :] = jnp.logaddexp(a, b)


def kernel(x):
    B, L, D = x.shape
    bl = 128
    lp = L + 16
    xp = jnp.pad(x[0], ((0, lp - L), (0, 0)))
    span_reps = pl.pallas_call(
        functools.partial(_span_body, bl=bl),
        grid=(L // bl,),
        in_specs=[pl.BlockSpec((lp, D), lambda i: (0, 0))],
        out_specs=pl.BlockSpec((bl, K, D), lambda i: (i, 0, 0)),
        out_shape=jax.ShapeDtypeStruct((L, K, D), x.dtype),
    )(xp)
    span_reps = span_reps[None]

    starts = jnp.arange(L, dtype=jnp.int32)
    ends = starts[:, None] + jnp.arange(K, dtype=jnp.int32)[None, :]
    span_idx = jnp.stack(
        [jnp.broadcast_to(starts[:, None], (L, K)), ends], axis=-1
    ).astype(jnp.int64)
    return span_reps, span_idx


# speedup vs baseline: 5.7612x; 1.0846x over previous
import functools

import jax
import jax.numpy as jnp
from jax.experimental import pallas as pl

K = 12
LOG2 = 0.6931471805599453


def _span_body(cur_ref, nxt_ref, out_ref, *, bl: int, length: int):
    i = pl.program_id(0)
    base = i * bl
    a = cur_ref[...]
    d = a.shape[-1]
    tail = nxt_ref[: K + 4, :]
    row = base + bl + jax.lax.broadcasted_iota(jnp.int32, tail.shape, 0)
    tail = jnp.where(row < length, tail, 0.0)
    w = jnp.concatenate([a, tail], axis=0)
    out_ref[0, :, 0, :] = a + LOG2
    for k in range(1, K):
        b = jax.lax.slice_in_dim(w, k, k + bl, axis=0)
        out_ref[0, :, k, :] = jnp.logaddexp(a, b)


def kernel(x):
    B, L, D = x.shape
    bl = 128
    n = L // bl
    x2 = x.reshape(L, D)
    span_reps = pl.pallas_call(
        functools.partial(_span_body, bl=bl, length=L),
        grid=(n,),
        in_specs=[
            pl.BlockSpec((bl, D), lambda i: (i, 0)),
            pl.BlockSpec((bl, D), lambda i: (jnp.minimum(i + 1, n - 1), 0)),
        ],
        out_specs=pl.BlockSpec((1, bl, K, D), lambda i: (0, i, 0, 0)),
        out_shape=jax.ShapeDtypeStruct((B, L, K, D), x.dtype),
    )(x2, x2)

    starts = jnp.arange(L, dtype=jnp.int32)
    ends = starts[:, None] + jnp.arange(K, dtype=jnp.int32)[None, :]
    span_idx = jnp.stack(
        [jnp.broadcast_to(starts[:, None], (L, K)), ends], axis=-1
    ).astype(jnp.int64)
    return span_reps, span_idx
